# Initial kernel scaffold; baseline (speedup 1.0000x reference)
import jax
import jax.numpy as jnp
from jax.experimental import pallas as pl


def kernel(input_ids, table):
    raise NotImplementedError("write your pallas kernel here")


# speedup vs baseline: 1.1092x; 1.1092x over previous
import functools
import math

import numpy as np
import jax
import jax.numpy as jnp
from jax import lax
from jax.experimental import pallas as pl
from jax.experimental.pallas import tpu as pltpu
from jax.experimental.pallas import tpu_sc as plsc

N_VOCAB = 100000
MAX_LENGTH = 4096
OUT_DIM = 128


def _make_pe(max_length, out_dim):
    position = np.arange(max_length, dtype=np.float32)[:, None]
    div_term = np.exp(
        np.arange(0, out_dim, 2, dtype=np.float32) * -(math.log(10000.0) / out_dim)
    )
    pe = np.zeros((max_length, out_dim), dtype=np.float32)
    pe[:, 0::2] = np.sin(position * div_term)
    pe[:, 1::2] = np.cos(position * div_term)
    return pe


_PE_NP = _make_pe(MAX_LENGTH, OUT_DIM)


@functools.cache
def _build(batch, seq, dim):
    info = plsc.get_sparse_core_info()
    nc, ns, lanes = info.num_cores, info.num_subcores, info.num_lanes
    nw = nc * ns
    assert seq % nw == 0
    ppw = seq // nw
    n_chunks = dim // lanes

    mesh = plsc.VectorSubcoreMesh(core_axis_name="c", subcore_axis_name="s")

    @functools.partial(
        pl.kernel,
        mesh=mesh,
        out_type=jax.ShapeDtypeStruct((batch, seq, dim), jnp.float32),
        scratch_types=[
            pltpu.VMEM((batch, ppw), jnp.int32),
            pltpu.VMEM((ppw, dim), jnp.float32),
            pltpu.VMEM((2, ppw, dim), jnp.float32),
            pltpu.SemaphoreType.DMA,
            pltpu.SemaphoreType.DMA,
        ],
    )
    def emb(idx_hbm, table_hbm, pe_hbm, out_hbm, idx_v, pe_v, rows_v, gsem, ssem):
        wid = lax.axis_index("s") * nc + lax.axis_index("c")
        pos0 = wid * ppw

        for b in range(batch):
            pltpu.sync_copy(idx_hbm.at[b, pl.ds(pos0, ppw)], idx_v.at[b])

        gathers = [None] * batch
        gathers[0] = pltpu.async_copy(
            table_hbm.at[idx_v.at[0]], rows_v.at[0], gsem
        )
        pltpu.sync_copy(pe_hbm.at[pl.ds(pos0, ppw)], pe_v)

        stores = [None] * batch
        for b in range(batch):
            buf = b % 2
            if b + 1 < batch:
                if stores[b - 1] is not None:
                    stores[b - 1].wait()
                gathers[b + 1] = pltpu.async_copy(
                    table_hbm.at[idx_v.at[b + 1]], rows_v.at[(b + 1) % 2], gsem
                )
            gathers[b].wait()

            def row_add(r, _, buf=buf):
                for c in range(n_chunks):
                    sl = pl.ds(c * lanes, lanes)
                    rows_v[buf, r, sl] = rows_v[buf, r, sl] + pe_v[r, sl]
                return _

            lax.fori_loop(0, ppw, row_add, 0)

            stores[b] = pltpu.async_copy(
                rows_v.at[buf], out_hbm.at[b, pl.ds(pos0, ppw)], ssem
            )
        stores[batch - 2].wait()
        stores[batch - 1].wait()

    return emb


def kernel(input_ids, table):
    batch, seq = input_ids.shape
    dim = table.shape[1]
    idx = input_ids.astype(jnp.int32)
    pe = jnp.asarray(_PE_NP[:seq], dtype=jnp.float32)
    return _build(batch, seq, dim)(idx, table, pe)
